# baseline (speedup 1.0000x reference)
import functools

import jax
import jax.numpy as jnp
from jax import lax
from jax.experimental import pallas as pl
from jax.experimental.pallas import tpu as pltpu
from jax.experimental.pallas import tpu_sc as plsc

H = 128
WEXT = 144
C = 40


def _linear_body(x_ref, w_ref, b_ref, o_ref):
    o_ref[...] = (
        jnp.dot(x_ref[...], w_ref[...], preferred_element_type=jnp.float32)
        + b_ref[...]
    )


def _linear(x, w, b, block_rows=0):
    n, k = x.shape
    m = w.shape[1]
    b2 = b.reshape(1, m)
    if not block_rows or n <= block_rows:
        return pl.pallas_call(
            _linear_body,
            out_shape=jax.ShapeDtypeStruct((n, m), jnp.float32),
        )(x, w, b2)
    assert n % block_rows == 0
    return pl.pallas_call(
        _linear_body,
        grid=(n // block_rows,),
        in_specs=[
            pl.BlockSpec((block_rows, k), lambda i: (i, 0)),
            pl.BlockSpec((k, m), lambda i: (0, 0)),
            pl.BlockSpec((1, m), lambda i: (0, 0)),
        ],
        out_specs=pl.BlockSpec((block_rows, m), lambda i: (i, 0)),
        out_shape=jax.ShapeDtypeStruct((n, m), jnp.float32),
    )(x, w, b2)


def _epilogue(x_old, nums, dens, biases, g, b):
    n = x_old.shape[0]
    nc = len(nums)

    def body(x_ref, *refs):
        num_refs = refs[:nc]
        den_refs = refs[nc:2 * nc]
        bias_refs = refs[2 * nc:3 * nc]
        g_ref, b_ref, o_ref = refs[3 * nc], refs[3 * nc + 1], refs[3 * nc + 2]
        y = x_ref[...]
        for nr, dr, br in zip(num_refs, den_refs, bias_refs):
            num = (nr[0] + nr[1])[:n]
            den = jnp.sum(dr[...], axis=0)[:n, None]
            y = y + num / (den + 1e-16) + br[...]
        mu = jnp.mean(y, axis=0, keepdims=True)
        var = jnp.mean((y - mu) ** 2, axis=0, keepdims=True)
        o_ref[...] = jnp.tanh((y - mu) / jnp.sqrt(var + 1e-5) * g_ref[...] + b_ref[...])

    args = [x_old] + list(nums) + list(dens) + \
        [bb.reshape(1, H) for bb in biases] + [g.reshape(1, H), b.reshape(1, H)]
    return pl.pallas_call(
        body, out_shape=jax.ShapeDtypeStruct((n, H), jnp.float32),
    )(*args)


def _pool_conv(x, bt, tok, p, nb):
    n, h = x.shape
    go = p['Wl'].shape[1]

    def body(x_ref, bt_ref, tok_ref, wl_ref, bl_ref, wr_ref, br_ref, att_ref,
             bias_ref, o_ref):
        x_ = x_ref[...]
        xl = jnp.dot(x_, wl_ref[...], preferred_element_type=jnp.float32) + bl_ref[...]
        xr = jnp.dot(tok_ref[...], wr_ref[...], preferred_element_type=jnp.float32) + br_ref[...]
        m = xl + xr
        lr = jnp.maximum(m, 0.2 * m)
        e = jnp.dot(lr, att_ref[...], preferred_element_type=jnp.float32)
        onehot = (bt_ref[...] == lax.broadcasted_iota(jnp.int32, (1, nb), 1)
                  ).astype(jnp.float32)
        emax = jnp.max(jnp.where(onehot > 0, e, -jnp.inf), axis=0, keepdims=True)
        emax = jnp.where(jnp.isfinite(emax), emax, 0.0)
        e_sh = lax.dot_general(onehot, emax, (((1,), (1,)), ((), ())))
        ex = jnp.exp(e - e_sh)
        den = lax.dot_general(onehot, ex, (((0,), (0,)), ((), ())))
        den_i = lax.dot_general(onehot, den, (((1,), (0,)), ((), ())))
        alpha = ex / (den_i + 1e-16)
        out = lax.dot_general(onehot, alpha * xl, (((0,), (0,)), ((), ())))
        o_ref[...] = out + bias_ref[...]

    return pl.pallas_call(
        body, out_shape=jax.ShapeDtypeStruct((nb, go), jnp.float32),
    )(x, bt.reshape(n, 1).astype(jnp.int32), tok.reshape(1, -1),
      p['Wl'], p['bl'].reshape(1, go), p['Wr'], p['br'].reshape(1, go),
      p['att'].reshape(go, 1), p['bias'].reshape(1, go))


def _mlp_head(cat, gm):
    nb = cat.shape[0]

    def rl(x, w1, b1, w2, b2, wp, bp):
        h = jnp.dot(jnp.tanh(jnp.dot(x, w1, preferred_element_type=jnp.float32) + b1),
                    w2, preferred_element_type=jnp.float32) + b2
        proj = x if wp is None else jnp.dot(x, wp, preferred_element_type=jnp.float32) + bp
        return proj + h

    def bn_tanh(x, g_, b_):
        mu = jnp.mean(x, axis=0, keepdims=True)
        var = jnp.mean((x - mu) ** 2, axis=0, keepdims=True)
        return jnp.tanh((x - mu) / jnp.sqrt(var + 1e-5) * g_ + b_)

    def body(cat_ref, w11, b11, w12, b12, wp1, bp1, g1, bb1,
             w21, b21, w22, b22, g2, bb2,
             w31, b31, w32, b32, wp3, bp3, o_ref):
        h = rl(cat_ref[...], w11[...], b11[...], w12[...], b12[...], wp1[...], bp1[...])
        h = bn_tanh(h, g1[...], bb1[...])
        h = rl(h, w21[...], b21[...], w22[...], b22[...], None, None)
        h = bn_tanh(h, g2[...], bb2[...])
        h = rl(h, w31[...], b31[...], w32[...], b32[...], wp3[...], bp3[...])
        o_ref[...] = h

    r1, r2, r3 = gm['rl1'], gm['rl2'], gm['rl3']
    v = lambda a: a.reshape(1, -1)
    args = [cat,
            r1['W1'], v(r1['b1']), r1['W2'], v(r1['b2']), r1['Wp'], v(r1['bp']),
            v(gm['bn1']['g']), v(gm['bn1']['b']),
            r2['W1'], v(r2['b1']), r2['W2'], v(r2['b2']),
            v(gm['bn2']['g']), v(gm['bn2']['b']),
            r3['W1'], v(r3['b1']), r3['W2'], v(r3['b2']), r3['Wp'], v(r3['bp'])]
    return pl.pallas_call(
        body, out_shape=jax.ShapeDtypeStruct((nb, 128), jnp.float32),
    )(*args)


_IOTA16 = None


def _splat_lane(vec, lane):
    idx = jnp.full((16, 1), lane, dtype=jnp.int32)
    return lax.gather(
        vec, idx,
        lax.GatherDimensionNumbers(offset_dims=(), collapsed_slice_dims=(0,),
                                   start_index_map=(0,)),
        (1,), mode=lax.GatherScatterMode.PROMISE_IN_BOUNDS)


def _edge_conv_sc_call(xl, xr, att, src, dst, has_e, eaw=None):
    n_dst = xr.shape[0]
    e_total = src.shape[0]
    rpt = ((-(-n_dst // 16)) + 63) // 64 * 64
    np_ = 16 * rpt
    et = e_total // 32
    assert et % C == 0 and et % 8 == 0
    nchunks = et // C
    ngroups = (C + 15) // 16
    zr = 128 if rpt % 128 == 0 else 64
    assert rpt % zr == 0

    mesh = plsc.VectorSubcoreMesh(core_axis_name="c", subcore_axis_name="s")
    scratch = [
        pltpu.VMEM((C,), jnp.int32),
        pltpu.VMEM((C,), jnp.int32),
        pltpu.VMEM((C, H), jnp.float32),
        pltpu.VMEM((C, H), jnp.float32),
        pltpu.VMEM((C, H), jnp.float32),
        pltpu.VMEM((C, H), jnp.float32),
        pltpu.VMEM((H,), jnp.float32),
        pltpu.VMEM((zr, H), jnp.float32),
        pltpu.VMEM((np_,), jnp.float32),
        pltpu.VMEM_SHARED((np_, H), jnp.float32),
        pltpu.SemaphoreType.DMA,
        pltpu.SemaphoreType.DMA,
        pltpu.SemaphoreType.DMA,
    ]

    def body(xl_hbm, xr_hbm, att_hbm, src_hbm, dst_hbm, eaw_hbm,
             num_hbm, den_hbm,
             sidx, didx, xlr, xrr, ear, wbuf, attv, zbuf, dent, acc,
             sm1, sm2, sm3):
        c = lax.axis_index("c")
        s = lax.axis_index("s")
        gid = c * 16 + s
        iota16 = lax.iota(jnp.int32, 16)
        zeros16 = jnp.zeros((16,), jnp.float32)

        def zrow(r, _):
            for k in range(H // 16):
                zbuf[r, pl.ds(k * 16, 16)] = zeros16
            return 0
        lax.fori_loop(0, zr, zrow, 0)

        def dzero(r, _):
            dent[pl.ds(r * 16, 16)] = zeros16
            return 0
        lax.fori_loop(0, np_ // 16, dzero, 0)

        for t in range(rpt // zr):
            pltpu.sync_copy(zbuf, acc.at[pl.ds(s * rpt + t * zr, zr)])
        pltpu.sync_copy(att_hbm, attv)
        plsc.subcore_barrier()

        def chunk(i, _):
            base = gid * et + i * C
            pltpu.sync_copy(src_hbm.at[pl.ds(base, C)], sidx)
            pltpu.sync_copy(dst_hbm.at[pl.ds(base, C)], didx)
            cp1 = pltpu.async_copy(xl_hbm.at[sidx], xlr, sm1)
            cp2 = pltpu.async_copy(xr_hbm.at[didx], xrr, sm2)
            if has_e:
                cp3 = pltpu.async_copy(eaw_hbm.at[pl.ds(base, C)], ear, sm3)
            cp1.wait()
            cp2.wait()
            if has_e:
                cp3.wait()

            for g in range(ngroups):
                gbase = min(g * 16, C - 16)
                novl = g * 16 - gbase
                rows = gbase + iota16

                accs = [zeros16, zeros16, zeros16, zeros16]
                for fc in range(8):
                    attvec = attv[pl.ds(fc * 16, 16)]
                    for k in range(16):
                        colv = jnp.full((16,), fc * 16 + k, jnp.int32)
                        a = plsc.load_gather(xlr, [rows, colv])
                        z = a + plsc.load_gather(xrr, [rows, colv])
                        if has_e:
                            z = z + plsc.load_gather(ear, [rows, colv])
                        lr = jnp.maximum(z, 0.2 * z)
                        accs[k % 4] = accs[k % 4] + _splat_lane(attvec, k) * lr
                e_g = (accs[0] + accs[1]) + (accs[2] + accs[3])
                ex_g = jnp.exp(e_g)
                dst16 = didx[pl.ds(gbase, 16)]
                if novl:
                    plsc.addupdate_scatter(dent, [dst16], ex_g,
                                           mask=iota16 >= novl)
                else:
                    plsc.addupdate_scatter(dent, [dst16], ex_g)
                for el in range(novl, 16):
                    r = gbase + el
                    exs = _splat_lane(ex_g, el)
                    for k in range(H // 16):
                        wbuf[r, pl.ds(k * 16, 16)] = exs * xlr[r, pl.ds(k * 16, 16)]

            pltpu.sync_copy(wbuf, acc.at[didx], add=True)
            return 0

        lax.fori_loop(0, nchunks, chunk, 0)
        plsc.subcore_barrier()

        pltpu.sync_copy(acc.at[pl.ds(s * rpt, rpt)],
                        num_hbm.at[c, pl.ds(s * rpt, rpt)])
        pltpu.sync_copy(dent, den_hbm.at[gid])

    if eaw is None:
        eaw = jnp.zeros((8, H), jnp.float32)
    return pl.kernel(
        body,
        out_type=[jax.ShapeDtypeStruct((2, np_, H), jnp.float32),
                  jax.ShapeDtypeStruct((32, np_), jnp.float32)],
        mesh=mesh,
        scratch_types=scratch,
        compiler_params=pltpu.CompilerParams(needs_layout_passes=False),
    )(xl, xr, att, src, dst, eaw)


def _gat_edge(xl, xr, att, ei, eaw=None):
    src = ei[0].astype(jnp.int32)
    dst = ei[1].astype(jnp.int32)
    return _edge_conv_sc_call(xl, xr, att, src, dst, eaw is not None, eaw)


def kernel(x_operation, x_machine, x_AGV, global_attr, edge_index_pred,
           edge_index_succ, edge_index_op_machine, edge_attr_op_machine,
           edge_index_machine_AGV, batch_operation, batch_machine, batch_AGV,
           params):
    pi = params['init']
    x_op = _linear(x_operation, pi['operation']['W'], pi['operation']['b'], 2000)
    x_m = _linear(x_machine, pi['machine']['W'], pi['machine']['b'])
    x_a = _linear(x_AGV, pi['AGV']['W'], pi['AGV']['b'])

    for lp in params['layers']:
        pp, ps, po, pa = lp['pred'], lp['succ'], lp['om'], lp['ma']
        w_op = jnp.concatenate(
            [pp['Wl'], pp['Wr'], ps['Wl'], ps['Wr'], po['Wl']], axis=1)
        b_op = jnp.concatenate(
            [pp['bl'], pp['br'], ps['bl'], ps['br'], po['bl']])
        y_op = _linear(x_op, w_op, b_op, 2000)
        xl_p, xr_p = y_op[:, 0:H], y_op[:, H:2 * H]
        xl_s, xr_s = y_op[:, 2 * H:3 * H], y_op[:, 3 * H:4 * H]
        xl_o = y_op[:, 4 * H:5 * H]

        w_m = jnp.concatenate([po['Wr'], pa['Wl']], axis=1)
        b_m = jnp.concatenate([po['br'], pa['bl']])
        y_m = _linear(x_m, w_m, b_m)
        xr_o, xl_a = y_m[:, 0:H], y_m[:, H:2 * H]
        xr_a = _linear(x_a, pa['Wr'], pa['br'])

        eaw = _linear(edge_attr_op_machine, po['We'],
                      jnp.zeros((H,), jnp.float32), 2000)

        num_p, den_p = _gat_edge(xl_p, xr_p, pp['att'], edge_index_pred)
        num_s, den_s = _gat_edge(xl_s, xr_s, ps['att'], edge_index_succ)
        num_o, den_o = _gat_edge(xl_o, xr_o, po['att'], edge_index_op_machine, eaw)
        num_a, den_a = _gat_edge(xl_a, xr_a, pa['att'], edge_index_machine_AGV)

        bn = lp['bn']
        x_op = _epilogue(x_op, [num_p, num_s], [den_p, den_s],
                         [pp['bias'], ps['bias']],
                         bn['operation']['g'], bn['operation']['b'])
        x_m = _epilogue(x_m, [num_o], [den_o], [po['bias']],
                        bn['machine']['g'], bn['machine']['b'])
        x_a = _epilogue(x_a, [num_a], [den_a], [pa['bias']],
                        bn['AGV']['g'], bn['AGV']['b'])

    mp = params['mix']
    nb = global_attr.shape[0]
    gd_op = _pool_conv(x_op, batch_operation, mp['tok']['operation'],
                       mp['conv']['operation'], nb)
    gd_m = _pool_conv(x_m, batch_machine, mp['tok']['machine'],
                      mp['conv']['machine'], nb)
    gd_a = _pool_conv(x_a, batch_AGV, mp['tok']['AGV'], mp['conv']['AGV'], nb)

    cat = jnp.concatenate([global_attr, gd_op, gd_m, gd_a], axis=1)
    h = _mlp_head(cat, mp['graph_mix'])
    return (x_op, x_m, x_a, gd_op, gd_m, gd_a, h)


# speedup vs baseline: 1.3778x; 1.3778x over previous
import functools

import jax
import jax.numpy as jnp
from jax import lax
from jax.experimental import pallas as pl
from jax.experimental.pallas import tpu as pltpu
from jax.experimental.pallas import tpu_sc as plsc

H = 128
WEXT = 144
C = 40


def _linear_body(x_ref, w_ref, b_ref, o_ref):
    o_ref[...] = (
        jnp.dot(x_ref[...], w_ref[...], preferred_element_type=jnp.float32)
        + b_ref[...]
    )


def _linear(x, w, b, block_rows=0):
    n, k = x.shape
    m = w.shape[1]
    b2 = b.reshape(1, m)
    if not block_rows or n <= block_rows:
        return pl.pallas_call(
            _linear_body,
            out_shape=jax.ShapeDtypeStruct((n, m), jnp.float32),
        )(x, w, b2)
    assert n % block_rows == 0
    return pl.pallas_call(
        _linear_body,
        grid=(n // block_rows,),
        in_specs=[
            pl.BlockSpec((block_rows, k), lambda i: (i, 0)),
            pl.BlockSpec((k, m), lambda i: (0, 0)),
            pl.BlockSpec((1, m), lambda i: (0, 0)),
        ],
        out_specs=pl.BlockSpec((block_rows, m), lambda i: (i, 0)),
        out_shape=jax.ShapeDtypeStruct((n, m), jnp.float32),
    )(x, w, b2)


def _epilogue(x_old, nums, dens, biases, g, b):
    n = x_old.shape[0]
    nc = len(nums)

    def body(x_ref, *refs):
        num_refs = refs[:nc]
        den_refs = refs[nc:2 * nc]
        bias_refs = refs[2 * nc:3 * nc]
        g_ref, b_ref, o_ref = refs[3 * nc], refs[3 * nc + 1], refs[3 * nc + 2]
        y = x_ref[...]
        for nr, dr, br in zip(num_refs, den_refs, bias_refs):
            num = (nr[0] + nr[1])[:n]
            den = jnp.sum(dr[...], axis=0)[:n, None]
            y = y + num / (den + 1e-16) + br[...]
        mu = jnp.mean(y, axis=0, keepdims=True)
        var = jnp.mean((y - mu) ** 2, axis=0, keepdims=True)
        o_ref[...] = jnp.tanh((y - mu) / jnp.sqrt(var + 1e-5) * g_ref[...] + b_ref[...])

    args = [x_old] + list(nums) + list(dens) + \
        [bb.reshape(1, H) for bb in biases] + [g.reshape(1, H), b.reshape(1, H)]
    return pl.pallas_call(
        body, out_shape=jax.ShapeDtypeStruct((n, H), jnp.float32),
    )(*args)


def _pool_conv(x, bt, tok, p, nb):
    n, h = x.shape
    go = p['Wl'].shape[1]

    def body(x_ref, bt_ref, tok_ref, wl_ref, bl_ref, wr_ref, br_ref, att_ref,
             bias_ref, o_ref):
        x_ = x_ref[...]
        xl = jnp.dot(x_, wl_ref[...], preferred_element_type=jnp.float32) + bl_ref[...]
        xr = jnp.dot(tok_ref[...], wr_ref[...], preferred_element_type=jnp.float32) + br_ref[...]
        m = xl + xr
        lr = jnp.maximum(m, 0.2 * m)
        e = jnp.dot(lr, att_ref[...], preferred_element_type=jnp.float32)
        onehot = (bt_ref[...] == lax.broadcasted_iota(jnp.int32, (1, nb), 1)
                  ).astype(jnp.float32)
        emax = jnp.max(jnp.where(onehot > 0, e, -jnp.inf), axis=0, keepdims=True)
        emax = jnp.where(jnp.isfinite(emax), emax, 0.0)
        e_sh = lax.dot_general(onehot, emax, (((1,), (1,)), ((), ())))
        ex = jnp.exp(e - e_sh)
        den = lax.dot_general(onehot, ex, (((0,), (0,)), ((), ())))
        den_i = lax.dot_general(onehot, den, (((1,), (0,)), ((), ())))
        alpha = ex / (den_i + 1e-16)
        out = lax.dot_general(onehot, alpha * xl, (((0,), (0,)), ((), ())))
        o_ref[...] = out + bias_ref[...]

    return pl.pallas_call(
        body, out_shape=jax.ShapeDtypeStruct((nb, go), jnp.float32),
    )(x, bt.reshape(n, 1).astype(jnp.int32), tok.reshape(1, -1),
      p['Wl'], p['bl'].reshape(1, go), p['Wr'], p['br'].reshape(1, go),
      p['att'].reshape(go, 1), p['bias'].reshape(1, go))


def _mlp_head(cat, gm):
    nb = cat.shape[0]

    def rl(x, w1, b1, w2, b2, wp, bp):
        h = jnp.dot(jnp.tanh(jnp.dot(x, w1, preferred_element_type=jnp.float32) + b1),
                    w2, preferred_element_type=jnp.float32) + b2
        proj = x if wp is None else jnp.dot(x, wp, preferred_element_type=jnp.float32) + bp
        return proj + h

    def bn_tanh(x, g_, b_):
        mu = jnp.mean(x, axis=0, keepdims=True)
        var = jnp.mean((x - mu) ** 2, axis=0, keepdims=True)
        return jnp.tanh((x - mu) / jnp.sqrt(var + 1e-5) * g_ + b_)

    def body(cat_ref, w11, b11, w12, b12, wp1, bp1, g1, bb1,
             w21, b21, w22, b22, g2, bb2,
             w31, b31, w32, b32, wp3, bp3, o_ref):
        h = rl(cat_ref[...], w11[...], b11[...], w12[...], b12[...], wp1[...], bp1[...])
        h = bn_tanh(h, g1[...], bb1[...])
        h = rl(h, w21[...], b21[...], w22[...], b22[...], None, None)
        h = bn_tanh(h, g2[...], bb2[...])
        h = rl(h, w31[...], b31[...], w32[...], b32[...], wp3[...], bp3[...])
        o_ref[...] = h

    r1, r2, r3 = gm['rl1'], gm['rl2'], gm['rl3']
    v = lambda a: a.reshape(1, -1)
    args = [cat,
            r1['W1'], v(r1['b1']), r1['W2'], v(r1['b2']), r1['Wp'], v(r1['bp']),
            v(gm['bn1']['g']), v(gm['bn1']['b']),
            r2['W1'], v(r2['b1']), r2['W2'], v(r2['b2']),
            v(gm['bn2']['g']), v(gm['bn2']['b']),
            r3['W1'], v(r3['b1']), r3['W2'], v(r3['b2']), r3['Wp'], v(r3['bp'])]
    return pl.pallas_call(
        body, out_shape=jax.ShapeDtypeStruct((nb, 128), jnp.float32),
    )(*args)


_IOTA16 = None


def _splat_lane(vec, lane):
    idx = jnp.full((16, 1), lane, dtype=jnp.int32)
    return lax.gather(
        vec, idx,
        lax.GatherDimensionNumbers(offset_dims=(), collapsed_slice_dims=(0,),
                                   start_index_map=(0,)),
        (1,), mode=lax.GatherScatterMode.PROMISE_IN_BOUNDS)


def _edge_conv_sc_call(xl, xr, att, src, dst, has_e, eaw=None):
    n_dst = xr.shape[0]
    e_total = src.shape[0]
    rpt = ((-(-n_dst // 16)) + 63) // 64 * 64
    np_ = 16 * rpt
    et = e_total // 32
    assert et % C == 0 and et % 8 == 0
    nchunks = et // C
    ngroups = (C + 15) // 16
    zr = 32
    assert rpt % zr == 0

    mesh = plsc.VectorSubcoreMesh(core_axis_name="c", subcore_axis_name="s")
    scratch = (
        [pltpu.VMEM((C,), jnp.int32)] * 6 +
        [pltpu.VMEM((C, H), jnp.float32)] * (6 + 2 * has_e) +
        [
            pltpu.VMEM((H,), jnp.float32),
            pltpu.VMEM((zr, H), jnp.float32),
            pltpu.VMEM((np_,), jnp.float32),
            pltpu.VMEM_SHARED((np_, H), jnp.float32),
        ] +
        [pltpu.SemaphoreType.DMA] * 6
    )

    def body(xl_hbm, xr_hbm, att_hbm, src_hbm, dst_hbm, eaw_hbm,
             num_hbm, den_hbm, *rest):
        SIDX, DIDX, DSCT = rest[0:2], rest[2:4], rest[4:6]
        XLR, XRR = rest[6:8], rest[8:10]
        if has_e:
            EAR, WBUF, k0 = rest[10:12], rest[12:14], 14
        else:
            EAR, WBUF, k0 = (None, None), rest[10:12], 12
        attv, zbuf, dent, acc = rest[k0:k0 + 4]
        ISEM, GSEM, SSEM = (rest[k0 + 4:k0 + 6], rest[k0 + 6:k0 + 8],
                            rest[k0 + 8:k0 + 10])
        c = lax.axis_index("c")
        s = lax.axis_index("s")
        gid = c * 16 + s
        iota16 = lax.iota(jnp.int32, 16)
        zeros16 = jnp.zeros((16,), jnp.float32)

        def zrow(r, _):
            for k in range(H // 16):
                zbuf[r, pl.ds(k * 16, 16)] = zeros16
            return 0
        lax.fori_loop(0, zr, zrow, 0)

        def dzero(r, _):
            dent[pl.ds(r * 16, 16)] = zeros16
            return 0
        lax.fori_loop(0, np_ // 16, dzero, 0)

        for t in range(rpt // zr):
            pltpu.sync_copy(zbuf, acc.at[pl.ds(s * rpt + t * zr, zr)])
        pltpu.sync_copy(att_hbm, attv)
        plsc.subcore_barrier()

        def base_of(ci):
            return jnp.minimum(gid * et + ci * C, e_total - C)

        def idx_descs(ci, b):
            base = base_of(ci)
            return (
                pltpu.make_async_copy(src_hbm.at[pl.ds(base, C)], SIDX[b], ISEM[b]),
                pltpu.make_async_copy(dst_hbm.at[pl.ds(base, C)], DIDX[b], ISEM[b]),
            )

        def gat_descs(ci, b):
            ds_ = [
                pltpu.make_async_copy(xl_hbm.at[SIDX[b]], XLR[b], GSEM[b]),
                pltpu.make_async_copy(xr_hbm.at[DIDX[b]], XRR[b], GSEM[b]),
            ]
            if has_e:
                ds_.append(pltpu.make_async_copy(
                    eaw_hbm.at[pl.ds(base_of(ci), C)], EAR[b], GSEM[b]))
            return ds_

        def sct_desc(b):
            return pltpu.make_async_copy(WBUF[b], acc.at[DSCT[b]], SSEM[b])

        def compute(b):
            for g in range(ngroups):
                gbase = min(g * 16, C - 16)
                novl = g * 16 - gbase
                rows = gbase + iota16

                def floop(fc, accs):
                    a0, a1, a2, a3 = accs
                    attvec = attv[pl.ds(fc * 16, 16)]
                    res = [a0, a1, a2, a3]
                    for k in range(16):
                        colv = jnp.full((16,), fc * 16 + k, jnp.int32)
                        z = (plsc.load_gather(XLR[b], [rows, colv])
                             + plsc.load_gather(XRR[b], [rows, colv]))
                        if has_e:
                            z = z + plsc.load_gather(EAR[b], [rows, colv])
                        lr = jnp.maximum(z, 0.2 * z)
                        res[k % 4] = res[k % 4] + _splat_lane(attvec, k) * lr
                    return tuple(res)

                accs = lax.fori_loop(0, 8, floop,
                                     (zeros16, zeros16, zeros16, zeros16))
                e_g = (accs[0] + accs[1]) + (accs[2] + accs[3])
                ex_g = jnp.exp(e_g)
                dst16 = DSCT[b][pl.ds(gbase, 16)]
                if novl:
                    plsc.addupdate_scatter(dent, [dst16], ex_g,
                                           mask=iota16 >= novl)
                else:
                    plsc.addupdate_scatter(dent, [dst16], ex_g)
                for el in range(novl, 16):
                    r = gbase + el
                    exs = _splat_lane(ex_g, el)
                    for k in range(H // 16):
                        WBUF[b][r, pl.ds(k * 16, 16)] = \
                            exs * XLR[b][r, pl.ds(k * 16, 16)]

        def step(ci, b, nb, first):
            for d in gat_descs(0, b):
                d.wait()
            for d in idx_descs(0, nb):
                d.wait()
            for d in gat_descs(ci + 1, nb):
                d.start()
            if not first:
                sct_desc(b).wait()
            for o in (0, 16, C - 16):
                DSCT[b][pl.ds(o, 16)] = DIDX[b][pl.ds(o, 16)]
            for d in idx_descs(ci + 2, b):
                d.start()
            compute(b)
            sct_desc(b).start(add=True)

        for d in idx_descs(0, 0):
            d.start()
        for d in idx_descs(1, 1):
            d.start()
        for d in idx_descs(0, 0):
            d.wait()
        for d in gat_descs(0, 0):
            d.start()

        step(0, 0, 1, True)
        if nchunks > 1:
            step(1, 1, 0, True)
        rem = nchunks - 2
        if rem > 0:
            def super_(j, _):
                ci = 2 + 2 * j
                step(ci, 0, 1, False)
                step(ci + 1, 1, 0, False)
                return 0
            lax.fori_loop(0, rem // 2, super_, 0)
            if rem % 2:
                step(nchunks - 1, (nchunks - 1) % 2, nchunks % 2, False)

        sct_desc((nchunks - 1) % 2).wait()
        if nchunks > 1:
            sct_desc((nchunks - 2) % 2).wait()
        for d in gat_descs(0, nchunks % 2):
            d.wait()
        for d in idx_descs(0, (nchunks + 1) % 2):
            d.wait()

        plsc.subcore_barrier()

        pltpu.sync_copy(acc.at[pl.ds(s * rpt, rpt)],
                        num_hbm.at[c, pl.ds(s * rpt, rpt)])
        pltpu.sync_copy(dent, den_hbm.at[gid])

    if eaw is None:
        eaw = jnp.zeros((8, H), jnp.float32)
    return pl.kernel(
        body,
        out_type=[jax.ShapeDtypeStruct((2, np_, H), jnp.float32),
                  jax.ShapeDtypeStruct((32, np_), jnp.float32)],
        mesh=mesh,
        scratch_types=scratch,
        compiler_params=pltpu.CompilerParams(needs_layout_passes=False),
    )(xl, xr, att, src, dst, eaw)


def _gat_edge(xl, xr, att, ei, eaw=None):
    src = ei[0].astype(jnp.int32)
    dst = ei[1].astype(jnp.int32)
    return _edge_conv_sc_call(xl, xr, att, src, dst, eaw is not None, eaw)


def kernel(x_operation, x_machine, x_AGV, global_attr, edge_index_pred,
           edge_index_succ, edge_index_op_machine, edge_attr_op_machine,
           edge_index_machine_AGV, batch_operation, batch_machine, batch_AGV,
           params):
    pi = params['init']
    x_op = _linear(x_operation, pi['operation']['W'], pi['operation']['b'], 2000)
    x_m = _linear(x_machine, pi['machine']['W'], pi['machine']['b'])
    x_a = _linear(x_AGV, pi['AGV']['W'], pi['AGV']['b'])

    for lp in params['layers']:
        pp, ps, po, pa = lp['pred'], lp['succ'], lp['om'], lp['ma']
        w_op = jnp.concatenate(
            [pp['Wl'], pp['Wr'], ps['Wl'], ps['Wr'], po['Wl']], axis=1)
        b_op = jnp.concatenate(
            [pp['bl'], pp['br'], ps['bl'], ps['br'], po['bl']])
        y_op = _linear(x_op, w_op, b_op, 2000)
        xl_p, xr_p = y_op[:, 0:H], y_op[:, H:2 * H]
        xl_s, xr_s = y_op[:, 2 * H:3 * H], y_op[:, 3 * H:4 * H]
        xl_o = y_op[:, 4 * H:5 * H]

        w_m = jnp.concatenate([po['Wr'], pa['Wl']], axis=1)
        b_m = jnp.concatenate([po['br'], pa['bl']])
        y_m = _linear(x_m, w_m, b_m)
        xr_o, xl_a = y_m[:, 0:H], y_m[:, H:2 * H]
        xr_a = _linear(x_a, pa['Wr'], pa['br'])

        eaw = _linear(edge_attr_op_machine, po['We'],
                      jnp.zeros((H,), jnp.float32), 2000)

        num_p, den_p = _gat_edge(xl_p, xr_p, pp['att'], edge_index_pred)
        num_s, den_s = _gat_edge(xl_s, xr_s, ps['att'], edge_index_succ)
        num_o, den_o = _gat_edge(xl_o, xr_o, po['att'], edge_index_op_machine, eaw)
        num_a, den_a = _gat_edge(xl_a, xr_a, pa['att'], edge_index_machine_AGV)

        bn = lp['bn']
        x_op = _epilogue(x_op, [num_p, num_s], [den_p, den_s],
                         [pp['bias'], ps['bias']],
                         bn['operation']['g'], bn['operation']['b'])
        x_m = _epilogue(x_m, [num_o], [den_o], [po['bias']],
                        bn['machine']['g'], bn['machine']['b'])
        x_a = _epilogue(x_a, [num_a], [den_a], [pa['bias']],
                        bn['AGV']['g'], bn['AGV']['b'])

    mp = params['mix']
    nb = global_attr.shape[0]
    gd_op = _pool_conv(x_op, batch_operation, mp['tok']['operation'],
                       mp['conv']['operation'], nb)
    gd_m = _pool_conv(x_m, batch_machine, mp['tok']['machine'],
                      mp['conv']['machine'], nb)
    gd_a = _pool_conv(x_a, batch_AGV, mp['tok']['AGV'], mp['conv']['AGV'], nb)

    cat = jnp.concatenate([global_attr, gd_op, gd_m, gd_a], axis=1)
    h = _mlp_head(cat, mp['graph_mix'])
    return (x_op, x_m, x_a, gd_op, gd_m, gd_a, h)


# speedup vs baseline: 4.7646x; 3.4582x over previous
import functools

import jax
import jax.numpy as jnp
from jax import lax
from jax.experimental import pallas as pl
from jax.experimental.pallas import tpu as pltpu
from jax.experimental.pallas import tpu_sc as plsc

H = 128
WEXT = 144
C = 40


def _linear_body(x_ref, w_ref, b_ref, o_ref):
    o_ref[...] = (
        jnp.dot(x_ref[...], w_ref[...], preferred_element_type=jnp.float32)
        + b_ref[...]
    )


def _linear(x, w, b, block_rows=0):
    n, k = x.shape
    m = w.shape[1]
    b2 = b.reshape(1, m)
    if not block_rows or n <= block_rows:
        return pl.pallas_call(
            _linear_body,
            out_shape=jax.ShapeDtypeStruct((n, m), jnp.float32),
        )(x, w, b2)
    assert n % block_rows == 0
    return pl.pallas_call(
        _linear_body,
        grid=(n // block_rows,),
        in_specs=[
            pl.BlockSpec((block_rows, k), lambda i: (i, 0)),
            pl.BlockSpec((k, m), lambda i: (0, 0)),
            pl.BlockSpec((1, m), lambda i: (0, 0)),
        ],
        out_specs=pl.BlockSpec((block_rows, m), lambda i: (i, 0)),
        out_shape=jax.ShapeDtypeStruct((n, m), jnp.float32),
    )(x, w, b2)


def _epilogue(x_old, nums, dens, invs, biases, g, b):
    n = x_old.shape[0]
    nc = len(nums)

    def body(x_ref, *refs):
        num_refs = refs[:nc]
        den_refs = refs[nc:2 * nc]
        inv_refs = refs[2 * nc:3 * nc]
        bias_refs = refs[3 * nc:4 * nc]
        g_ref, b_ref, o_ref = refs[4 * nc], refs[4 * nc + 1], refs[4 * nc + 2]
        y = x_ref[...]
        for nr, dr, ir, br in zip(num_refs, den_refs, inv_refs, bias_refs):
            num = (nr[0] + nr[1])[:n] * ir[...]
            den = jnp.sum(dr[...], axis=0)[:n, None]
            y = y + num / (den + 1e-16) + br[...]
        mu = jnp.mean(y, axis=0, keepdims=True)
        var = jnp.mean((y - mu) ** 2, axis=0, keepdims=True)
        o_ref[...] = jnp.tanh((y - mu) / jnp.sqrt(var + 1e-5) * g_ref[...] + b_ref[...])

    args = [x_old] + list(nums) + list(dens) + \
        [iv.reshape(1, H) for iv in invs] + \
        [bb.reshape(1, H) for bb in biases] + [g.reshape(1, H), b.reshape(1, H)]
    return pl.pallas_call(
        body, out_shape=jax.ShapeDtypeStruct((n, H), jnp.float32),
    )(*args)


def _pool_conv(x, bt, tok, p, nb):
    n, h = x.shape
    go = p['Wl'].shape[1]

    def body(x_ref, bt_ref, tok_ref, wl_ref, bl_ref, wr_ref, br_ref, att_ref,
             bias_ref, o_ref):
        x_ = x_ref[...]
        xl = jnp.dot(x_, wl_ref[...], preferred_element_type=jnp.float32) + bl_ref[...]
        xr = jnp.dot(tok_ref[...], wr_ref[...], preferred_element_type=jnp.float32) + br_ref[...]
        m = xl + xr
        lr = jnp.maximum(m, 0.2 * m)
        e = jnp.dot(lr, att_ref[...], preferred_element_type=jnp.float32)
        onehot = (bt_ref[...] == lax.broadcasted_iota(jnp.int32, (1, nb), 1)
                  ).astype(jnp.float32)
        emax = jnp.max(jnp.where(onehot > 0, e, -jnp.inf), axis=0, keepdims=True)
        emax = jnp.where(jnp.isfinite(emax), emax, 0.0)
        e_sh = lax.dot_general(onehot, emax, (((1,), (1,)), ((), ())))
        ex = jnp.exp(e - e_sh)
        den = lax.dot_general(onehot, ex, (((0,), (0,)), ((), ())))
        den_i = lax.dot_general(onehot, den, (((1,), (0,)), ((), ())))
        alpha = ex / (den_i + 1e-16)
        out = lax.dot_general(onehot, alpha * xl, (((0,), (0,)), ((), ())))
        o_ref[...] = out + bias_ref[...]

    return pl.pallas_call(
        body, out_shape=jax.ShapeDtypeStruct((nb, go), jnp.float32),
    )(x, bt.reshape(n, 1).astype(jnp.int32), tok.reshape(1, -1),
      p['Wl'], p['bl'].reshape(1, go), p['Wr'], p['br'].reshape(1, go),
      p['att'].reshape(go, 1), p['bias'].reshape(1, go))


def _mlp_head(cat, gm):
    nb = cat.shape[0]

    def rl(x, w1, b1, w2, b2, wp, bp):
        h = jnp.dot(jnp.tanh(jnp.dot(x, w1, preferred_element_type=jnp.float32) + b1),
                    w2, preferred_element_type=jnp.float32) + b2
        proj = x if wp is None else jnp.dot(x, wp, preferred_element_type=jnp.float32) + bp
        return proj + h

    def bn_tanh(x, g_, b_):
        mu = jnp.mean(x, axis=0, keepdims=True)
        var = jnp.mean((x - mu) ** 2, axis=0, keepdims=True)
        return jnp.tanh((x - mu) / jnp.sqrt(var + 1e-5) * g_ + b_)

    def body(cat_ref, w11, b11, w12, b12, wp1, bp1, g1, bb1,
             w21, b21, w22, b22, g2, bb2,
             w31, b31, w32, b32, wp3, bp3, o_ref):
        h = rl(cat_ref[...], w11[...], b11[...], w12[...], b12[...], wp1[...], bp1[...])
        h = bn_tanh(h, g1[...], bb1[...])
        h = rl(h, w21[...], b21[...], w22[...], b22[...], None, None)
        h = bn_tanh(h, g2[...], bb2[...])
        h = rl(h, w31[...], b31[...], w32[...], b32[...], wp3[...], bp3[...])
        o_ref[...] = h

    r1, r2, r3 = gm['rl1'], gm['rl2'], gm['rl3']
    v = lambda a: a.reshape(1, -1)
    args = [cat,
            r1['W1'], v(r1['b1']), r1['W2'], v(r1['b2']), r1['Wp'], v(r1['bp']),
            v(gm['bn1']['g']), v(gm['bn1']['b']),
            r2['W1'], v(r2['b1']), r2['W2'], v(r2['b2']),
            v(gm['bn2']['g']), v(gm['bn2']['b']),
            r3['W1'], v(r3['b1']), r3['W2'], v(r3['b2']), r3['Wp'], v(r3['bp'])]
    return pl.pallas_call(
        body, out_shape=jax.ShapeDtypeStruct((nb, 128), jnp.float32),
    )(*args)


def _edge_conv_sc_call(xl, xr, att, src, dst, has_e, eaw=None):
    n_dst = xr.shape[0]
    e_total = src.shape[0]
    rpt = ((-(-n_dst // 16)) + 63) // 64 * 64
    np_ = 16 * rpt
    et = e_total // 32
    assert et % C == 0 and et % 8 == 0
    nchunks = et // C
    ngroups = (C + 15) // 16
    zr = 32
    assert rpt % zr == 0

    mesh = plsc.VectorSubcoreMesh(core_axis_name="c", subcore_axis_name="s")
    scratch = (
        [pltpu.VMEM((C,), jnp.int32)] * 6 +
        [pltpu.VMEM((C, H), jnp.float32)] * (6 + 2 * has_e) +
        [
            pltpu.VMEM((H,), jnp.float32),
            pltpu.VMEM((zr, H), jnp.float32),
            pltpu.VMEM((np_,), jnp.float32),
            pltpu.VMEM((C,), jnp.float32),
            pltpu.VMEM_SHARED((np_, H), jnp.float32),
        ] +
        [pltpu.SemaphoreType.DMA] * 6
    )

    def body(xl_hbm, xr_hbm, att_hbm, src_hbm, dst_hbm, eaw_hbm,
             num_hbm, den_hbm, *rest):
        SIDX, DIDX, DSCT = rest[0:2], rest[2:4], rest[4:6]
        XLR, XRR = rest[6:8], rest[8:10]
        if has_e:
            EAR, WBUF, k0 = rest[10:12], rest[12:14], 14
        else:
            EAR, WBUF, k0 = (None, None), rest[10:12], 12
        attv, zbuf, dent, exbuf, acc = rest[k0:k0 + 5]
        ISEM, GSEM, SSEM = (rest[k0 + 5:k0 + 7], rest[k0 + 7:k0 + 9],
                            rest[k0 + 9:k0 + 11])
        c = lax.axis_index("c")
        s = lax.axis_index("s")
        gid = c * 16 + s
        iota16 = lax.iota(jnp.int32, 16)
        zeros16 = jnp.zeros((16,), jnp.float32)

        def zrow(r, _):
            for k in range(H // 16):
                zbuf[r, pl.ds(k * 16, 16)] = zeros16
            return 0
        lax.fori_loop(0, zr, zrow, 0)

        def dzero(r, _):
            dent[pl.ds(r * 16, 16)] = zeros16
            return 0
        lax.fori_loop(0, np_ // 16, dzero, 0)

        for t in range(rpt // zr):
            pltpu.sync_copy(zbuf, acc.at[pl.ds(s * rpt + t * zr, zr)])
        pltpu.sync_copy(att_hbm, attv)
        plsc.subcore_barrier()

        def base_of(ci):
            return jnp.minimum(gid * et + ci * C, e_total - C)

        def idx_descs(ci, b):
            base = base_of(ci)
            return (
                pltpu.make_async_copy(src_hbm.at[pl.ds(base, C)], SIDX[b], ISEM[b]),
                pltpu.make_async_copy(dst_hbm.at[pl.ds(base, C)], DIDX[b], ISEM[b]),
            )

        def gat_descs(ci, b):
            ds_ = [
                pltpu.make_async_copy(xl_hbm.at[SIDX[b]], XLR[b], GSEM[b]),
                pltpu.make_async_copy(xr_hbm.at[DIDX[b]], XRR[b], GSEM[b]),
            ]
            if has_e:
                ds_.append(pltpu.make_async_copy(
                    eaw_hbm.at[pl.ds(base_of(ci), C)], EAR[b], GSEM[b]))
            return ds_

        def sct_desc(b):
            return pltpu.make_async_copy(WBUF[b], acc.at[DSCT[b]], SSEM[b])

        def compute(b):
            svecs = [attv[pl.ds(fc * 16, 16)] for fc in range(8)]
            lane0 = iota16 == 0

            def epair(p, carry):
                for eo in range(2):
                    e = 2 * p + eo
                    uls = []
                    acc_a = acc_b = zeros16
                    for fc in range(8):
                        ul = XLR[b][e, pl.ds(fc * 16, 16)]
                        z = ul + XRR[b][e, pl.ds(fc * 16, 16)]
                        if has_e:
                            z = z + EAR[b][e, pl.ds(fc * 16, 16)]
                        lr = jnp.maximum(z, 0.2 * z)
                        if fc % 2:
                            acc_a = acc_a + svecs[fc] * lr
                        else:
                            acc_b = acc_b + svecs[fc] * lr
                        uls.append(ul)
                    tot = jnp.sum(acc_a + acc_b)
                    exv = jnp.exp(jnp.full((16,), tot, jnp.float32))
                    for fc in range(8):
                        WBUF[b][e, pl.ds(fc * 16, 16)] = exv * uls[fc]
                    plsc.store_scatter(exbuf, [jnp.full((16,), e, jnp.int32)],
                                       exv, mask=lane0)
                return carry

            lax.fori_loop(0, C // 2, epair, 0)

            for g in range(ngroups):
                gbase = min(g * 16, C - 16)
                novl = g * 16 - gbase
                ex_g = exbuf[pl.ds(gbase, 16)]
                dst16 = DSCT[b][pl.ds(gbase, 16)]
                if novl:
                    plsc.addupdate_scatter(dent, [dst16], ex_g,
                                           mask=iota16 >= novl)
                else:
                    plsc.addupdate_scatter(dent, [dst16], ex_g)

        def step(ci, b, nb, first):
            for d in gat_descs(0, b):
                d.wait()
            for d in idx_descs(0, nb):
                d.wait()
            for d in gat_descs(ci + 1, nb):
                d.start()
            if not first:
                sct_desc(b).wait()
            for o in (0, 16, C - 16):
                DSCT[b][pl.ds(o, 16)] = DIDX[b][pl.ds(o, 16)]
            for d in idx_descs(ci + 2, b):
                d.start()
            compute(b)
            sct_desc(b).start(add=True)

        for d in idx_descs(0, 0):
            d.start()
        for d in idx_descs(1, 1):
            d.start()
        for d in idx_descs(0, 0):
            d.wait()
        for d in gat_descs(0, 0):
            d.start()

        step(0, 0, 1, True)
        if nchunks > 1:
            step(1, 1, 0, True)
        rem = nchunks - 2
        if rem > 0:
            def super_(j, _):
                ci = 2 + 2 * j
                step(ci, 0, 1, False)
                step(ci + 1, 1, 0, False)
                return 0
            lax.fori_loop(0, rem // 2, super_, 0)
            if rem % 2:
                step(nchunks - 1, (nchunks - 1) % 2, nchunks % 2, False)

        sct_desc((nchunks - 1) % 2).wait()
        if nchunks > 1:
            sct_desc((nchunks - 2) % 2).wait()
        for d in gat_descs(0, nchunks % 2):
            d.wait()
        for d in idx_descs(0, (nchunks + 1) % 2):
            d.wait()

        plsc.subcore_barrier()

        pltpu.sync_copy(acc.at[pl.ds(s * rpt, rpt)],
                        num_hbm.at[c, pl.ds(s * rpt, rpt)])
        pltpu.sync_copy(dent, den_hbm.at[gid])

    if eaw is None:
        eaw = jnp.zeros((8, H), jnp.float32)
    return pl.kernel(
        body,
        out_type=[jax.ShapeDtypeStruct((2, np_, H), jnp.float32),
                  jax.ShapeDtypeStruct((32, np_), jnp.float32)],
        mesh=mesh,
        scratch_types=scratch,
        compiler_params=pltpu.CompilerParams(needs_layout_passes=False),
    )(xl, xr, att, src, dst, eaw)


def _gat_edge(xl, xr, att, ei, eaw=None):
    src = ei[0].astype(jnp.int32)
    dst = ei[1].astype(jnp.int32)
    return _edge_conv_sc_call(xl, xr, att, src, dst, eaw is not None, eaw)


def kernel(x_operation, x_machine, x_AGV, global_attr, edge_index_pred,
           edge_index_succ, edge_index_op_machine, edge_attr_op_machine,
           edge_index_machine_AGV, batch_operation, batch_machine, batch_AGV,
           params):
    pi = params['init']
    x_op = _linear(x_operation, pi['operation']['W'], pi['operation']['b'], 2000)
    x_m = _linear(x_machine, pi['machine']['W'], pi['machine']['b'])
    x_a = _linear(x_AGV, pi['AGV']['W'], pi['AGV']['b'])

    for lp in params['layers']:
        pp, ps, po, pa = lp['pred'], lp['succ'], lp['om'], lp['ma']
        a_p, a_s = jnp.abs(pp['att']), jnp.abs(ps['att'])
        a_o, a_a = jnp.abs(po['att']), jnp.abs(pa['att'])
        w_op = jnp.concatenate(
            [pp['Wl'] * a_p, pp['Wr'] * a_p, ps['Wl'] * a_s, ps['Wr'] * a_s,
             po['Wl'] * a_o], axis=1)
        b_op = jnp.concatenate(
            [pp['bl'] * a_p, pp['br'] * a_p, ps['bl'] * a_s, ps['br'] * a_s,
             po['bl'] * a_o])
        y_op = _linear(x_op, w_op, b_op, 2000)
        xl_p, xr_p = y_op[:, 0:H], y_op[:, H:2 * H]
        xl_s, xr_s = y_op[:, 2 * H:3 * H], y_op[:, 3 * H:4 * H]
        xl_o = y_op[:, 4 * H:5 * H]

        w_m = jnp.concatenate([po['Wr'] * a_o, pa['Wl'] * a_a], axis=1)
        b_m = jnp.concatenate([po['br'] * a_o, pa['bl'] * a_a])
        y_m = _linear(x_m, w_m, b_m)
        xr_o, xl_a = y_m[:, 0:H], y_m[:, H:2 * H]
        xr_a = _linear(x_a, pa['Wr'] * a_a, pa['br'] * a_a)

        eaw = _linear(edge_attr_op_machine, po['We'] * a_o,
                      jnp.zeros((H,), jnp.float32), 2000)

        num_p, den_p = _gat_edge(xl_p, xr_p, jnp.sign(pp['att']), edge_index_pred)
        num_s, den_s = _gat_edge(xl_s, xr_s, jnp.sign(ps['att']), edge_index_succ)
        num_o, den_o = _gat_edge(xl_o, xr_o, jnp.sign(po['att']),
                                 edge_index_op_machine, eaw)
        num_a, den_a = _gat_edge(xl_a, xr_a, jnp.sign(pa['att']),
                                 edge_index_machine_AGV)

        inv = lambda a: jnp.where(a > 0, 1.0 / a, 0.0)
        bn = lp['bn']
        x_op = _epilogue(x_op, [num_p, num_s], [den_p, den_s],
                         [inv(a_p), inv(a_s)],
                         [pp['bias'], ps['bias']],
                         bn['operation']['g'], bn['operation']['b'])
        x_m = _epilogue(x_m, [num_o], [den_o], [inv(a_o)], [po['bias']],
                        bn['machine']['g'], bn['machine']['b'])
        x_a = _epilogue(x_a, [num_a], [den_a], [inv(a_a)], [pa['bias']],
                        bn['AGV']['g'], bn['AGV']['b'])

    mp = params['mix']
    nb = global_attr.shape[0]
    gd_op = _pool_conv(x_op, batch_operation, mp['tok']['operation'],
                       mp['conv']['operation'], nb)
    gd_m = _pool_conv(x_m, batch_machine, mp['tok']['machine'],
                      mp['conv']['machine'], nb)
    gd_a = _pool_conv(x_a, batch_AGV, mp['tok']['AGV'], mp['conv']['AGV'], nb)

    cat = jnp.concatenate([global_attr, gd_op, gd_m, gd_a], axis=1)
    h = _mlp_head(cat, mp['graph_mix'])
    return (x_op, x_m, x_a, gd_op, gd_m, gd_a, h)
